# baseline (speedup 1.0000x reference)
import functools

import jax
import jax.numpy as jnp
from jax.experimental import pallas as pl
from jax.experimental.pallas import tpu as pltpu

_NBUF = 2


def _pick_bm(n: int) -> int:
    best = 8
    for bm in range(8, 257, 8):
        if n % bm == 0:
            best = bm
    return best


def _gcn_body(x_ref, adj_hbm, w1_ref, b1_ref, w2_ref, b2_ref, out_ref,
              s_ref, h_ref, cache_ref, bufs_ref, sems,
              *, bm: int, g: int, k: int):
    p = pl.program_id(0)
    m = pl.program_id(1)
    k2 = 2 * k
    nfetch = 2 * g - k

    skip = (p == 1) & ((m % 2) == 1) & (m < k2)
    skipped_before = jnp.where(p == 0, 0, jnp.minimum(m, k2) // 2)
    f_t = p * g + m - skipped_before

    def block_of(j):
        f1 = j - g
        return jnp.where(j < g, j,
                         jnp.where(f1 < k, 2 * f1, f1 + k))

    def issue(j):
        @pl.when(j < nfetch)
        def _():
            b = block_of(j)
            for nb in range(_NBUF):
                @pl.when((j % _NBUF) == nb)
                def _(nb=nb):
                    pltpu.make_async_copy(
                        adj_hbm.at[pl.ds(b * bm, bm), :],
                        bufs_ref.at[nb],
                        sems.at[nb],
                    ).start()

    @pl.when((p == 0) & (m == 0))
    def _():
        for j in range(_NBUF - 1):
            issue(jnp.int32(j))

    @pl.when(jnp.logical_not(skip))
    def _():
        issue(f_t + (_NBUF - 1))
        b = block_of(f_t)
        for nb in range(_NBUF):
            @pl.when((f_t % _NBUF) == nb)
            def _(nb=nb):
                pltpu.make_async_copy(
                    adj_hbm.at[pl.ds(b * bm, bm), :],
                    bufs_ref.at[nb],
                    sems.at[nb],
                ).wait()

    @pl.when((p == 0) & (m == 0))
    def _():
        s1 = jnp.dot(x_ref[...].astype(jnp.bfloat16),
                     w1_ref[...].astype(jnp.bfloat16),
                     preferred_element_type=jnp.float32)
        s_ref[...] = s1.astype(jnp.bfloat16)

    cached_blk = ((m % 2) == 1) & (m < k2)

    @pl.when(p == 0)
    def _():
        for nb in range(_NBUF):
            @pl.when((f_t % _NBUF) == nb)
            def _(nb=nb):
                a_bf = bufs_ref[nb].astype(jnp.bfloat16)
                acc = jnp.dot(a_bf, s_ref[...],
                              preferred_element_type=jnp.float32)
                h = jnp.maximum(acc + b1_ref[...], 0.0)
                h_ref[pl.ds(m * bm, bm), :] = h.astype(jnp.bfloat16)

                @pl.when(cached_blk)
                def _(a_bf=a_bf):
                    cache_ref[pl.ds((m // 2) * bm, bm), :] = a_bf

    @pl.when((p == 1) & (m == 0))
    def _():
        s2 = jnp.dot(h_ref[...], w2_ref[...].astype(jnp.bfloat16),
                     preferred_element_type=jnp.float32)
        s_ref[...] = s2.astype(jnp.bfloat16)

    @pl.when((p == 1) & jnp.logical_not(skip))
    def _():
        for nb in range(_NBUF):
            @pl.when((f_t % _NBUF) == nb)
            def _(nb=nb):
                a_bf = bufs_ref[nb].astype(jnp.bfloat16)
                acc = jnp.dot(a_bf, s_ref[...],
                              preferred_element_type=jnp.float32)
                out_ref[...] = acc + b2_ref[...]

    @pl.when((p == 1) & skip)
    def _():
        a_bf = cache_ref[pl.ds((m // 2) * bm, bm), :]
        acc = jnp.dot(a_bf, s_ref[...], preferred_element_type=jnp.float32)
        out_ref[...] = acc + b2_ref[...]


@jax.jit
def kernel(x, adj, W1, b1, W2, b2):
    n, nfeat = x.shape
    nhid = W1.shape[1]
    nout = W2.shape[1]
    bm = _pick_bm(n)
    g = n // bm

    vmem_budget = 64 * 1024 * 1024
    fixed = (_NBUF * bm * n * 4
             + n * nfeat * 4
             + 2 * n * nhid * 2
             + 9 * 1024 * 1024)
    blk_bytes = bm * n * 2
    k = max(0, min(g // 2, (vmem_budget - fixed) // blk_bytes))

    b1r = b1.reshape(1, nhid)
    b2r = b2.reshape(1, nout)

    return pl.pallas_call(
        functools.partial(_gcn_body, bm=bm, g=g, k=k),
        grid=(2, g),
        in_specs=[
            pl.BlockSpec((n, nfeat), lambda p, m: (0, 0)),
            pl.BlockSpec(memory_space=pltpu.MemorySpace.HBM),
            pl.BlockSpec((nfeat, nhid), lambda p, m: (0, 0)),
            pl.BlockSpec((1, nhid), lambda p, m: (0, 0)),
            pl.BlockSpec((nhid, nout), lambda p, m: (0, 0)),
            pl.BlockSpec((1, nout), lambda p, m: (0, 0)),
        ],
        out_specs=pl.BlockSpec((bm, nout),
                               lambda p, m: (jnp.where(p == 1, m, 0), 0)),
        out_shape=jax.ShapeDtypeStruct((n, nout), jnp.float32),
        scratch_shapes=[
            pltpu.VMEM((n, nhid), jnp.bfloat16),
            pltpu.VMEM((n, nhid), jnp.bfloat16),
            pltpu.VMEM((max(k * bm, 8), n), jnp.bfloat16),
            pltpu.VMEM((_NBUF, bm, n), jnp.float32),
            pltpu.SemaphoreType.DMA((_NBUF,)),
        ],
        compiler_params=pltpu.CompilerParams(
            dimension_semantics=("arbitrary", "arbitrary"),
            vmem_limit_bytes=63 * 1024 * 1024,
        ),
    )(x, adj, W1, b1r, W2, b2r)


# speedup vs baseline: 1.0510x; 1.0510x over previous
import functools

import jax
import jax.numpy as jnp
from jax.experimental import pallas as pl
from jax.experimental.pallas import tpu as pltpu

_NBUF = 3


def _pick_bm(n: int) -> int:
    best = 8
    for bm in range(8, 257, 8):
        if n % bm == 0:
            best = bm
    return best


def _chunks(bm: int):
    c1 = ((bm // 2 + 7) // 8) * 8
    if c1 <= 0 or c1 >= bm:
        return [(0, bm)]
    return [(0, c1), (c1, bm - c1)]


def _gcn_body(x_ref, adj_hbm, w1_ref, b1_ref, w2_ref, b2_ref, out_ref,
              s_ref, h_ref, cache_ref, bufs_ref, sems,
              *, bm: int, g: int, k: int):
    p = pl.program_id(0)
    m = pl.program_id(1)
    k2 = 2 * k
    nfetch = 2 * g - k

    skip = (p == 1) & ((m % 2) == 1) & (m < k2)
    skipped_before = jnp.where(p == 0, 0, jnp.minimum(m, k2) // 2)
    f_t = p * g + m - skipped_before

    def block_of(j):
        f1 = j - g
        return jnp.where(j < g, j,
                         jnp.where(f1 < k, 2 * f1, f1 + k))

    def issue(j):
        @pl.when(j < nfetch)
        def _():
            b = block_of(j)
            for nb in range(_NBUF):
                @pl.when((j % _NBUF) == nb)
                def _(nb=nb):
                    pltpu.make_async_copy(
                        adj_hbm.at[pl.ds(b * bm, bm), :],
                        bufs_ref.at[nb],
                        sems.at[nb],
                    ).start()

    @pl.when((p == 0) & (m == 0))
    def _():
        for j in range(_NBUF - 1):
            issue(jnp.int32(j))

    @pl.when(jnp.logical_not(skip))
    def _():
        issue(f_t + (_NBUF - 1))
        b = block_of(f_t)
        for nb in range(_NBUF):
            @pl.when((f_t % _NBUF) == nb)
            def _(nb=nb):
                pltpu.make_async_copy(
                    adj_hbm.at[pl.ds(b * bm, bm), :],
                    bufs_ref.at[nb],
                    sems.at[nb],
                ).wait()

    @pl.when((p == 0) & (m == 0))
    def _():
        s1 = jnp.dot(x_ref[...].astype(jnp.bfloat16),
                     w1_ref[...].astype(jnp.bfloat16),
                     preferred_element_type=jnp.float32)
        s_ref[...] = s1.astype(jnp.bfloat16)

    cached_blk = ((m % 2) == 1) & (m < k2)

    @pl.when(p == 0)
    def _():
        for nb in range(_NBUF):
            @pl.when((f_t % _NBUF) == nb)
            def _(nb=nb):
                for r0, rs in _chunks(bm):
                    a_bf = bufs_ref[nb, r0:r0 + rs, :].astype(jnp.bfloat16)
                    acc = jnp.dot(a_bf, s_ref[...],
                                  preferred_element_type=jnp.float32)
                    h = jnp.maximum(acc + b1_ref[...], 0.0)
                    h_ref[pl.ds(m * bm + r0, rs), :] = h.astype(jnp.bfloat16)

                    @pl.when(cached_blk)
                    def _(a_bf=a_bf, r0=r0, rs=rs):
                        cache_ref[pl.ds((m // 2) * bm + r0, rs), :] = a_bf

    @pl.when((p == 1) & (m == 0))
    def _():
        s2 = jnp.dot(h_ref[...], w2_ref[...].astype(jnp.bfloat16),
                     preferred_element_type=jnp.float32)
        s_ref[...] = s2.astype(jnp.bfloat16)

    @pl.when((p == 1) & jnp.logical_not(skip))
    def _():
        for nb in range(_NBUF):
            @pl.when((f_t % _NBUF) == nb)
            def _(nb=nb):
                for r0, rs in _chunks(bm):
                    a_bf = bufs_ref[nb, r0:r0 + rs, :].astype(jnp.bfloat16)
                    acc = jnp.dot(a_bf, s_ref[...],
                                  preferred_element_type=jnp.float32)
                    out_ref[pl.ds(r0, rs), :] = acc + b2_ref[...]

    @pl.when((p == 1) & skip)
    def _():
        a_bf = cache_ref[pl.ds((m // 2) * bm, bm), :]
        acc = jnp.dot(a_bf, s_ref[...], preferred_element_type=jnp.float32)
        out_ref[...] = acc + b2_ref[...]


@jax.jit
def kernel(x, adj, W1, b1, W2, b2):
    n, nfeat = x.shape
    nhid = W1.shape[1]
    nout = W2.shape[1]
    bm = _pick_bm(n)
    g = n // bm

    vmem_budget = 68 * 1024 * 1024
    fixed = (_NBUF * bm * n * 4
             + n * nfeat * 4
             + 2 * n * nhid * 2
             + 9 * 1024 * 1024)
    blk_bytes = bm * n * 2
    k = max(0, min(g // 2, (vmem_budget - fixed) // blk_bytes))

    b1r = b1.reshape(1, nhid)
    b2r = b2.reshape(1, nout)

    return pl.pallas_call(
        functools.partial(_gcn_body, bm=bm, g=g, k=k),
        grid=(2, g),
        in_specs=[
            pl.BlockSpec((n, nfeat), lambda p, m: (0, 0)),
            pl.BlockSpec(memory_space=pltpu.MemorySpace.HBM),
            pl.BlockSpec((nfeat, nhid), lambda p, m: (0, 0)),
            pl.BlockSpec((1, nhid), lambda p, m: (0, 0)),
            pl.BlockSpec((nhid, nout), lambda p, m: (0, 0)),
            pl.BlockSpec((1, nout), lambda p, m: (0, 0)),
        ],
        out_specs=pl.BlockSpec((bm, nout),
                               lambda p, m: (jnp.where(p == 1, m, 0), 0)),
        out_shape=jax.ShapeDtypeStruct((n, nout), jnp.float32),
        scratch_shapes=[
            pltpu.VMEM((n, nhid), jnp.bfloat16),
            pltpu.VMEM((n, nhid), jnp.bfloat16),
            pltpu.VMEM((max(k * bm, 8), n), jnp.bfloat16),
            pltpu.VMEM((_NBUF, bm, n), jnp.float32),
            pltpu.SemaphoreType.DMA((_NBUF,)),
        ],
        compiler_params=pltpu.CompilerParams(
            dimension_semantics=("arbitrary", "arbitrary"),
            vmem_limit_bytes=63 * 1024 * 1024,
        ),
    )(x, adj, W1, b1r, W2, b2r)
